# baseline (speedup 1.0000x reference)
import functools

import jax
import jax.numpy as jnp
from jax import lax
from jax.experimental import pallas as pl
from jax.experimental.pallas import tpu as pltpu
from jax.experimental.pallas import tpu_sc as plsc

_BATCH = 16
_N_FREQ = 8192
_N_LOW = 2048
_ROW_IN = _N_FREQ * 2
_ROW_OUT = _N_LOW * 2
_FLAT_IN = _BATCH * _ROW_IN
_FLAT_OUT = _BATCH * _ROW_OUT
_NUM_WORKERS = 32
_CHUNK = _FLAT_OUT // _NUM_WORKERS

_mesh = plsc.VectorSubcoreMesh(core_axis_name="c", subcore_axis_name="s")


@functools.partial(
    pl.kernel,
    out_type=jax.ShapeDtypeStruct((_FLAT_OUT,), jnp.float32),
    mesh=_mesh,
    scratch_types=[pltpu.VMEM((_CHUNK,), jnp.float32)],
)
def _gather_low_sc(in_hbm, out_hbm, buf):
    wid = lax.axis_index("s") * 2 + lax.axis_index("c")
    batch = wid // 2
    half = wid % 2
    src = batch * _ROW_IN + half * _CHUNK
    dst = batch * _ROW_OUT + half * _CHUNK
    pltpu.sync_copy(in_hbm.at[pl.ds(src, _CHUNK)], buf)
    pltpu.sync_copy(buf, out_hbm.at[pl.ds(dst, _CHUNK)])


def kernel(full_freq_info):
    flat = full_freq_info.reshape(_FLAT_IN)
    low = _gather_low_sc(flat).reshape(_BATCH, _N_LOW, 2)
    return (full_freq_info, low)
